# baseline (speedup 1.0000x reference)
import functools

import jax
import jax.numpy as jnp
from jax import lax
from jax.experimental import pallas as pl
from jax.experimental.pallas import tpu as pltpu
from jax.experimental.pallas import tpu_sc as plsc

N_VOCAB = 49408
N_EMBD = 768
N_TOKEN = 77
BATCH = 1024

NC = 2
NS = 16
NW = NC * NS
LANES = 16

FLAT = BATCH * N_TOKEN
ROWS_W = FLAT // NW
CHUNK = 16
NCH = ROWS_W // CHUNK
DGROUPS = N_EMBD // LANES

_mesh = plsc.VectorSubcoreMesh(
    core_axis_name="c", subcore_axis_name="s", num_cores=NC, num_subcores=NS
)


@functools.partial(
    pl.kernel,
    out_type=jax.ShapeDtypeStruct((FLAT, N_EMBD), jnp.float32),
    mesh=_mesh,
    scratch_types=[
        pltpu.VMEM((ROWS_W,), jnp.int32),
        pltpu.VMEM((N_TOKEN * N_EMBD,), jnp.float32),
        pltpu.VMEM((2, CHUNK, N_EMBD), jnp.float32),
        pltpu.SemaphoreType.DMA,
        pltpu.SemaphoreType.DMA,
        pltpu.SemaphoreType.DMA,
        pltpu.SemaphoreType.DMA,
    ],
)
def _emb_lookup(tok_hbm, tab_hbm, pos_hbm, out_hbm,
                idx_v, pos_v, rows_v, g0, g1, s0, s1):
    gsem = (g0, g1)
    ssem = (s0, s1)
    wid = lax.axis_index("s") * NC + lax.axis_index("c")
    base = wid * ROWS_W

    pltpu.sync_copy(tok_hbm.at[pl.ds(base, ROWS_W)], idx_v)
    pltpu.sync_copy(pos_hbm, pos_v)

    def gather_desc(c, b):
        return pltpu.make_async_copy(
            tab_hbm.at[idx_v.at[pl.ds(c * CHUNK, CHUNK)]], rows_v.at[b], gsem[b]
        )

    def scatter_desc(c, b):
        return pltpu.make_async_copy(
            rows_v.at[b], out_hbm.at[pl.ds(base + c * CHUNK, CHUNK)], ssem[b]
        )

    gather_desc(0, 0).start()

    def pair_body(jj, carry):
        for b in range(2):
            j = jj * 2 + b
            bn = 1 - b

            @pl.when(j >= 1)
            def _():
                scatter_desc(j - 1, bn).wait()

            @pl.when(j + 1 < NCH)
            def _():
                gather_desc(j + 1, bn).start()

            gather_desc(j, b).wait()
            row0 = base + j * CHUNK
            p = row0 // BATCH
            pbase = p * N_EMBD
            for g in range(DGROUPS // 8):
                xs = [pos_v[pl.ds(pbase + (8 * g + d) * LANES, LANES)]
                      for d in range(8)]
                for i in range(CHUNK):
                    for d in range(8):
                        plsc.addupdate(
                            rows_v.at[b, i,
                                      pl.ds((8 * g + d) * LANES, LANES)],
                            xs[d])

            scatter_desc(j, b).start()
        return carry

    lax.fori_loop(0, NCH // 2, pair_body, 0)

    scatter_desc(NCH - 1, (NCH - 1) % 2).wait()


def kernel(tokens, token_embedding, position_value):
    tok_t = tokens.astype(jnp.int32).T.reshape(-1)
    pos = position_value.reshape(-1)
    out_t = _emb_lookup(tok_t, token_embedding, pos)
    return out_t.reshape(N_TOKEN, BATCH, N_EMBD).transpose(1, 0, 2)


# speedup vs baseline: 1.1901x; 1.1901x over previous
import functools

import jax
import jax.numpy as jnp
from jax import lax
from jax.experimental import pallas as pl
from jax.experimental.pallas import tpu as pltpu
from jax.experimental.pallas import tpu_sc as plsc

N_VOCAB = 49408
N_EMBD = 768
N_TOKEN = 77
BATCH = 1024

NC = 2
NS = 16
NW = NC * NS
LANES = 16

FLAT = BATCH * N_TOKEN
ROWS_W = FLAT // NW
CHUNK = 16
NCH = ROWS_W // CHUNK
DGROUPS = N_EMBD // LANES

_mesh = plsc.VectorSubcoreMesh(
    core_axis_name="c", subcore_axis_name="s", num_cores=NC, num_subcores=NS
)


@functools.partial(
    pl.kernel,
    out_type=jax.ShapeDtypeStruct((FLAT, N_EMBD), jnp.float32),
    mesh=_mesh,
    scratch_types=[
        pltpu.VMEM((ROWS_W,), jnp.int32),
        pltpu.VMEM((N_TOKEN * N_EMBD,), jnp.float32),
        pltpu.VMEM((2, CHUNK, N_EMBD), jnp.float32),
        pltpu.SemaphoreType.DMA,
        pltpu.SemaphoreType.DMA,
        pltpu.SemaphoreType.DMA,
        pltpu.SemaphoreType.DMA,
    ],
)
def _emb_lookup(tok_hbm, tab_hbm, pos_hbm, out_hbm,
                idx_v, pos_v, rows_v, g0, g1, s0, s1):
    gsem = (g0, g1)
    ssem = (s0, s1)
    wid = lax.axis_index("s") * NC + lax.axis_index("c")
    base = wid * ROWS_W

    pltpu.sync_copy(tok_hbm.at[pl.ds(base, ROWS_W)], idx_v)
    pltpu.sync_copy(pos_hbm, pos_v)

    def gather_desc(c, b):
        return pltpu.make_async_copy(
            tab_hbm.at[idx_v.at[pl.ds(c * CHUNK, CHUNK)]], rows_v.at[b], gsem[b]
        )

    def scatter_desc(c, b):
        return pltpu.make_async_copy(
            rows_v.at[b], out_hbm.at[pl.ds(base + c * CHUNK, CHUNK)], ssem[b]
        )

    gather_desc(0, 0).start()

    def pair_body(jj, carry):
        for b in range(2):
            j = jj * 2 + b
            bn = 1 - b

            @pl.when(j >= 1)
            def _():
                scatter_desc(j - 1, bn).wait()

            @pl.when(j + 1 < NCH)
            def _():
                gather_desc(j + 1, bn).start()

            gather_desc(j, b).wait()
            row0 = base + j * CHUNK

            def row_body(i, carry):
                p = (row0 + i) // BATCH
                pbase = p * N_EMBD
                ngrp = DGROUPS // 8
                xs = [pos_v[pl.ds(pbase + d * LANES, LANES)] for d in range(8)]
                for g in range(1, ngrp + 1):
                    if g < ngrp:
                        ys = [pos_v[pl.ds(pbase + (8 * g + d) * LANES, LANES)]
                              for d in range(8)]
                    for d in range(8):
                        plsc.addupdate(
                            rows_v.at[b, i,
                                      pl.ds((8 * (g - 1) + d) * LANES, LANES)],
                            xs[d])
                    if g < ngrp:
                        xs = ys
                return carry

            lax.fori_loop(0, CHUNK, row_body, 0)
            scatter_desc(j, b).start()
        return carry

    lax.fori_loop(0, NCH // 2, pair_body, 0)

    scatter_desc(NCH - 1, (NCH - 1) % 2).wait()


def kernel(tokens, token_embedding, position_value):
    tok_t = tokens.astype(jnp.int32).T.reshape(-1)
    pos = position_value.reshape(-1)
    out_t = _emb_lookup(tok_t, token_embedding, pos)
    return out_t.reshape(N_TOKEN, BATCH, N_EMBD).transpose(1, 0, 2)
